# baseline (speedup 1.0000x reference)
import jax
import jax.numpy as jnp
from jax.experimental import pallas as pl
from jax.experimental.pallas import tpu as pltpu

_LANE = 128


def _round_up(x, m):
    return ((x + m - 1) // m) * m


def _mlp_body(x_ref, w1_ref, b1_ref, w2_ref, b2_ref, w3_ref, b3_ref, o_ref):
    x = x_ref[...]
    cm = (((1,), (1,)), ((), ()))
    h = jax.lax.dot_general(x, w1_ref[...], cm,
                            preferred_element_type=jnp.float32)
    h = jnp.maximum(h + b1_ref[...], 0.0)
    h = jax.lax.dot_general(h, w2_ref[...], cm,
                            preferred_element_type=jnp.float32)
    h = jnp.maximum(h + b2_ref[...], 0.0)
    logits = jax.lax.dot_general(h, w3_ref[...], cm,
                                 preferred_element_type=jnp.float32)
    logits = logits + b3_ref[...]

    m = jnp.max(logits, axis=1, keepdims=True)
    shifted = logits - m
    lse = jnp.log(jnp.sum(jnp.exp(shifted), axis=1, keepdims=True))
    o_ref[...] = (shifted - lse).astype(o_ref.dtype)


def kernel(x, w1, b1, w2, b2, w3, b3, *, block_batch=8192):
    B, n_in = x.shape
    h1, h2, n_out = w1.shape[0], w2.shape[0], w3.shape[0]

    BT = max(_LANE, min(block_batch, _round_up((B + 1) // 2, _LANE)))
    Bp = _round_up(B, BT)
    if Bp != B:
        x = jnp.pad(x, ((0, Bp - B), (0, 0)))
    grid = (Bp // BT,)

    b1r, b2r, b3r = (b.reshape(1, -1) for b in (b1, b2, b3))

    flops = 2 * Bp * (n_in * h1 + h1 * h2 + h2 * n_out)
    transcendentals = Bp * (n_out + 1)
    bytes_accessed = (Bp * n_in * 4 + Bp * n_out * 4
                      + (w1.size + w2.size + w3.size
                         + b1.size + b2.size + b3.size) * 4)

    const = lambda i: (0, 0)
    out = pl.pallas_call(
        _mlp_body,
        out_shape=jax.ShapeDtypeStruct((Bp, n_out), jnp.float32),
        grid=grid,
        in_specs=[
            pl.BlockSpec((BT, n_in), lambda i: (i, 0)),
            pl.BlockSpec(w1.shape, const), pl.BlockSpec(b1r.shape, const),
            pl.BlockSpec(w2.shape, const), pl.BlockSpec(b2r.shape, const),
            pl.BlockSpec(w3.shape, const), pl.BlockSpec(b3r.shape, const),
        ],
        out_specs=pl.BlockSpec((BT, n_out), lambda i: (i, 0)),
        compiler_params=pltpu.CompilerParams(
            dimension_semantics=("parallel",)),
        cost_estimate=pl.CostEstimate(
            flops=flops,
            transcendentals=transcendentals,
            bytes_accessed=bytes_accessed),
    )(x, w1, b1r, w2, b2r, w3, b3r)

    return out[:B]
